# baseline (speedup 1.0000x reference)
import functools

import jax
import jax.numpy as jnp
from jax import lax
from jax.experimental import pallas as pl
from jax.experimental.pallas import tpu as pltpu
from jax.experimental.pallas import tpu_sc as plsc

NUM_CLASS = 100000
BATCH = 4096
CTX_DIM = 512
N_CLS_CTX = 4
PROMPT_LEN = 4
SEQ = 77
SUFFIX_LEN = SEQ - (2 * PROMPT_LEN + 1 + N_CLS_CTX)

NC, NS = 2, 16
NW = NC * NS
BPW = BATCH // NW
CHUNK = 16
NCHUNK = BPW // CHUNK

ROW_PRE_P = 1
ROW_CLS = ROW_PRE_P + PROMPT_LEN
ROW_SUF_P = ROW_CLS + N_CLS_CTX
ROW_SUFFIX = ROW_SUF_P + PROMPT_LEN
HEAD = ROW_CLS
TAIL = SEQ - ROW_SUF_P


def _body(label_hbm, cls_hbm, tp_hbm, pp_hbm, sfp_hbm, sfx_hbm, out_hbm,
          idx_v, tmpl_v, cls_v, gsem, ssem):
    wid = lax.axis_index("s") * NC + lax.axis_index("c")
    base = wid * BPW

    pltpu.sync_copy(label_hbm.at[pl.ds(wid * NCHUNK, NCHUNK)], idx_v)
    pltpu.sync_copy(tp_hbm, tmpl_v.at[pl.ds(0, 1)])
    pltpu.sync_copy(pp_hbm, tmpl_v.at[pl.ds(ROW_PRE_P, PROMPT_LEN)])
    pltpu.sync_copy(sfp_hbm, tmpl_v.at[pl.ds(ROW_SUF_P, PROMPT_LEN)])
    pltpu.sync_copy(sfx_hbm, tmpl_v.at[pl.ds(ROW_SUFFIX, SUFFIX_LEN)])

    def chunk_body(c, carry):
        pltpu.async_copy(cls_hbm.at[idx_v.at[c]], cls_v, gsem).wait()
        handles = []
        for e in range(CHUNK):
            b = base + c * CHUNK + e
            handles.append(pltpu.async_copy(
                tmpl_v.at[pl.ds(0, HEAD)], out_hbm.at[b, pl.ds(0, HEAD)],
                ssem))
            handles.append(pltpu.async_copy(
                cls_v.at[e], out_hbm.at[b, pl.ds(ROW_CLS, N_CLS_CTX)],
                ssem))
            handles.append(pltpu.async_copy(
                tmpl_v.at[pl.ds(ROW_SUF_P, TAIL)],
                out_hbm.at[b, pl.ds(ROW_SUF_P, TAIL)], ssem))
        for h in handles:
            h.wait()
        return carry

    lax.fori_loop(0, NCHUNK, chunk_body, 0)


@functools.partial(jax.jit, donate_argnums=())
def _run(label2, cls_ctx, tp, pp, sfp, sfx):
    mesh = plsc.VectorSubcoreMesh(
        core_axis_name="c", subcore_axis_name="s",
        num_cores=NC, num_subcores=NS)
    return pl.kernel(
        _body,
        out_type=jax.ShapeDtypeStruct((BATCH, SEQ, CTX_DIM), jnp.float32),
        mesh=mesh,
        scratch_types=[
            pltpu.VMEM((NCHUNK, CHUNK), jnp.int32),
            pltpu.VMEM((SEQ, CTX_DIM), jnp.float32),
            pltpu.VMEM((CHUNK, N_CLS_CTX, CTX_DIM), jnp.float32),
            pltpu.SemaphoreType.DMA,
            pltpu.SemaphoreType.DMA,
        ],
        compiler_params=pltpu.CompilerParams(use_tc_tiling_on_sc=False),
        name="prompt_learner_sc",
    )(label2, cls_ctx, tp, pp, sfp, sfx)


def kernel(label, cls_ctx, token_prefix, token_suffix, prefix_prompt,
           suffix_prompt):
    label2 = label.reshape(NW * NCHUNK, CHUNK).astype(jnp.int32)
    tp = token_prefix.reshape(1, CTX_DIM)
    pp = prefix_prompt.reshape(PROMPT_LEN, CTX_DIM)
    sfp = suffix_prompt.reshape(PROMPT_LEN, CTX_DIM)
    sfx = token_suffix.reshape(SUFFIX_LEN, CTX_DIM)
    return _run(label2, cls_ctx, tp, pp, sfp, sfx)
